# baseline (speedup 1.0000x reference)
import jax
import jax.numpy as jnp
from jax.experimental import pallas as pl

B = 512
NPER = 16
N = B * NPER
HID = 64
TIME = 64
NF = 10
NL = 2
HEADS = 8
MAXZ = 100
EDIM = NF * 2 * 3 + 6

G = 8
TN = G * NPER
NT = B // G


def _sig(x):
    return 1.0 / (1.0 + jnp.exp(-x))


def _silu(x):
    return x * _sig(x)


def _ln(x, g, b):
    m = jnp.mean(x, axis=-1, keepdims=True)
    v = jnp.mean((x - m) ** 2, axis=-1, keepdims=True)
    return (x - m) / jnp.sqrt(v + 1e-5) * g + b


def _weight_list(params):
    out = []

    def add(name, a, as_row=False):
        if as_row:
            a = a.reshape(1, -1)
        out.append((name, a))

    add('emb', params['emb'])
    add('na_w', params['na']['w'])
    add('na_b', params['na']['b'], True)
    add('eln_g', params['eln_g'], True)
    add('eln_b', params['eln_b'], True)
    for i, lp in enumerate(params['layers']):
        p = f'l{i}_'
        add(p + 'g1', lp['g1'], True)
        add(p + 'b1', lp['b1'], True)
        for nm in ('q', 'k', 'v'):
            add(p + nm + 'w', lp[nm]['w'])
            add(p + nm + 'b', lp[nm]['b'], True)
        add(p + 'ew', lp['e']['w'])
        add(p + 'skw', lp['skip']['w'])
        add(p + 'skb', lp['skip']['b'], True)
        for gate in ('ga', 'gf'):
            for mm in ('m1', 'm2', 'm3'):
                add(p + gate + mm + 'w', lp[gate][mm]['w'])
                add(p + gate + mm + 'b', lp[gate][mm]['b'], True)
        add(p + 'g2', lp['g2'], True)
        add(p + 'b2', lp['b2'], True)
        add(p + 'f1w', lp['f1']['w'])
        add(p + 'f1b', lp['f1']['b'], True)
        add(p + 'f2w', lp['f2']['w'])
        add(p + 'f2b', lp['f2']['b'], True)
    for nm in ('lr1', 'lr2', 'fr1', 'fr2'):
        add(nm + 'w', params[nm]['w'])
        add(nm + 'b', params[nm]['b'], True)
    add('lr3w', params['lr3']['w'])
    add('fr3w', params['fr3']['w'])
    return out


def _mlp_gate(u, v, W, pre):
    z = jnp.concatenate([u, v, u - v], axis=-1)
    h1 = _silu(jnp.dot(z, W[pre + 'm1w'], preferred_element_type=jnp.float32)
               + W[pre + 'm1b'])
    h2 = _silu(jnp.dot(h1, W[pre + 'm2w'], preferred_element_type=jnp.float32)
               + W[pre + 'm2b'])
    a = _sig(jnp.dot(h2, W[pre + 'm3w'], preferred_element_type=jnp.float32)
             + W[pre + 'm3b'])
    return a * u + (1.0 - a) * v


def _make_body(names):
    idx = {nm: k for k, nm in enumerate(names)}

    def body(at_ref, t_ref, fr_ref, lrp_ref, *rest):
        wrefs = rest[:len(names)]
        out_lat_ref, out_fc_ref = rest[len(names):]
        W = {nm: wrefs[idx[nm]][...] for nm in names}

        f32 = jnp.float32

        tcol = at_ref[...]
        zi = jax.lax.broadcasted_iota(jnp.int32, (TN, MAXZ), 1)
        oh = (tcol - 1 == zi).astype(f32)
        h_type = jnp.dot(oh, W['emb'], preferred_element_type=f32)
        tb = jnp.broadcast_to(t_ref[...][:, None, :],
                              (G, NPER, TIME)).reshape(TN, TIME)
        h = jnp.dot(jnp.concatenate([h_type, tb], axis=-1), W['na_w'],
                    preferred_element_type=f32) + W['na_b']

        fr = fr_ref[...]
        freqs = 2.0 * jnp.pi * jax.lax.broadcasted_iota(
            jnp.int32, (1, NF), 1).astype(f32)
        angn = jnp.concatenate(
            [fr[:, d:d + 1] * freqs for d in range(3)], axis=-1)
        sn = jnp.sin(angn)
        cn = jnp.cos(angn)
        pj = jnp.concatenate([sn, cn], axis=-1).reshape(G, NPER, 60)
        qi = jnp.concatenate([cn, cn], axis=-1).reshape(G, NPER, 60)
        rj = jnp.concatenate([cn, sn], axis=-1).reshape(G, NPER, 60)
        ti = jnp.concatenate([sn, -sn], axis=-1).reshape(G, NPER, 60)
        trig = (pj[:, None, :, :] * qi[:, :, None, :]
                - rj[:, None, :, :] * ti[:, :, None, :])
        lrb = jnp.broadcast_to(lrp_ref[...][:, None, None, :],
                               (G, NPER, NPER, 6))
        ef = jnp.concatenate([trig, lrb], axis=-1)
        ef = _ln(ef, W['eln_g'], W['eln_b'])
        ef2 = ef.reshape(G * NPER * NPER, EDIM)

        hrow = jax.lax.broadcasted_iota(jnp.int32, (HEADS * HID, HEADS), 0)
        hcol = jax.lax.broadcasted_iota(jnp.int32, (HEADS * HID, HEADS), 1)
        hsel = (hrow // HID == hcol).astype(f32) * 0.125
        trow = jax.lax.broadcasted_iota(jnp.int32, (HEADS, HEADS * HID), 0)
        tcol2 = jax.lax.broadcasted_iota(jnp.int32, (HEADS, HEADS * HID), 1)
        hselT = (tcol2 // HID == trow).astype(f32)
        drow = jax.lax.broadcasted_iota(jnp.int32, (HEADS * HID, HID), 0)
        dcol = jax.lax.broadcasted_iota(jnp.int32, (HEADS * HID, HID), 1)
        hmean = (drow % HID == dcol).astype(f32) / float(HEADS)

        for li in range(NL):
            p = f'l{li}_'
            h = _ln(h, W[p + 'g1'], W[p + 'b1'])
            q = jnp.dot(h, W[p + 'qw'], preferred_element_type=f32) + W[p + 'qb']
            k = jnp.dot(h, W[p + 'kw'], preferred_element_type=f32) + W[p + 'kb']
            v = jnp.dot(h, W[p + 'vw'], preferred_element_type=f32) + W[p + 'vb']
            ee = jnp.dot(ef2, W[p + 'ew'], preferred_element_type=f32)

            q3 = q.reshape(G, NPER, HEADS * HID)
            k3 = k.reshape(G, NPER, HEADS * HID)
            v3 = v.reshape(G, NPER, HEADS * HID)
            ee4 = ee.reshape(G, NPER, NPER, HEADS * HID)
            kj = k3[:, :, None, :] + ee4
            tt = q3[:, None, :, :] * kj
            a2 = jnp.dot(tt.reshape(G * NPER * NPER, HEADS * HID), hsel,
                         preferred_element_type=f32,
                         precision=jax.lax.Precision.HIGHEST)
            a4 = a2.reshape(G, NPER, NPER, HEADS)
            mx = jnp.max(a4, axis=1, keepdims=True)
            ex = jnp.exp(a4 - mx)
            s = jnp.sum(ex, axis=1, keepdims=True)
            al = ex / (s + 1e-16)
            al_big = jnp.dot(al.reshape(G * NPER * NPER, HEADS), hselT,
                             preferred_element_type=f32,
                             precision=jax.lax.Precision.HIGHEST)
            msg = ((v3[:, :, None, :] + ee4)
                   * al_big.reshape(G, NPER, NPER, HEADS * HID))
            out = jnp.sum(msg, axis=1)
            attn = jnp.dot(out.reshape(TN, HEADS * HID), hmean,
                           preferred_element_type=f32,
                           precision=jax.lax.Precision.HIGHEST)
            vat = attn + jnp.dot(h, W[p + 'skw'],
                                 preferred_element_type=f32) + W[p + 'skb']
            h = _mlp_gate(h, vat, W, p + 'ga')
            h = _ln(h, W[p + 'g2'], W[p + 'b2'])
            ffn = jnp.dot(_silu(jnp.dot(h, W[p + 'f1w'],
                                        preferred_element_type=f32)
                                + W[p + 'f1b']),
                          W[p + 'f2w'], preferred_element_type=f32) + W[p + 'f2b']
            h = _mlp_gate(h, ffn, W, p + 'gf')

        gf = jnp.sum(h.reshape(G, NPER, HID), axis=1) / float(NPER)
        lat = _silu(jnp.dot(gf, W['lr1w'], preferred_element_type=f32)
                    + W['lr1b'])
        lat = _silu(jnp.dot(lat, W['lr2w'], preferred_element_type=f32)
                    + W['lr2b'])
        lat = jnp.dot(lat, W['lr3w'], preferred_element_type=f32)
        fc = _silu(jnp.dot(h, W['fr1w'], preferred_element_type=f32)
                   + W['fr1b'])
        fc = _silu(jnp.dot(fc, W['fr2w'], preferred_element_type=f32)
                   + W['fr2b'])
        fc = jnp.dot(fc, W['fr3w'], preferred_element_type=f32)
        out_lat_ref[...] = lat
        out_fc_ref[...] = fc

    return body


def kernel(t, atom_types, frac_coords, lattices_rep, num_atoms, node2graph,
           lattices_mat, cemb, guide_indicator, params):
    wl = _weight_list(params)
    names = [nm for nm, _ in wl]
    warrs = [a.astype(jnp.float32) for _, a in wl]

    at2 = atom_types.reshape(N, 1)

    in_specs = [
        pl.BlockSpec((TN, 1), lambda i: (i, 0)),
        pl.BlockSpec((G, TIME), lambda i: (i, 0)),
        pl.BlockSpec((TN, 3), lambda i: (i, 0)),
        pl.BlockSpec((G, 6), lambda i: (i, 0)),
    ]
    for a in warrs:
        in_specs.append(pl.BlockSpec(a.shape, lambda i: (0, 0)))

    out_shape = (
        jax.ShapeDtypeStruct((B, 6), jnp.float32),
        jax.ShapeDtypeStruct((N, 3), jnp.float32),
    )
    out_specs = (
        pl.BlockSpec((G, 6), lambda i: (i, 0)),
        pl.BlockSpec((TN, 3), lambda i: (i, 0)),
    )

    lat, fc = pl.pallas_call(
        _make_body(names),
        grid=(NT,),
        in_specs=in_specs,
        out_specs=out_specs,
        out_shape=out_shape,
    )(at2, t, frac_coords, lattices_rep, *warrs)
    return (lat, fc)


# speedup vs baseline: 2.2954x; 2.2954x over previous
import jax
import jax.numpy as jnp
from jax.experimental import pallas as pl

B = 512
NPER = 16
N = B * NPER
HID = 64
TIME = 64
NF = 10
NL = 2
HEADS = 8
MAXZ = 100
EDIM = NF * 2 * 3 + 6

G = 8
TN = G * NPER
NT = B // G


def _sig(x):
    return 1.0 / (1.0 + jnp.exp(-x))


def _silu(x):
    return x * _sig(x)


def _ln(x, g, b):
    m = jnp.mean(x, axis=-1, keepdims=True)
    v = jnp.mean((x - m) ** 2, axis=-1, keepdims=True)
    return (x - m) / jnp.sqrt(v + 1e-5) * g + b


def _weight_list(params):
    ll = jnp.arange(HEADS * HID)
    hperm = (ll % HEADS) * HID + ll // HEADS
    out = []

    def add(name, a, as_row=False):
        if as_row:
            a = a.reshape(1, -1)
        out.append((name, a))

    def add_p(name, a, as_row=False):
        a = a[hperm] if a.ndim == 1 else a[:, hperm]
        add(name, a, as_row)

    add('emb', params['emb'])
    add('na_w', params['na']['w'])
    add('na_b', params['na']['b'], True)
    add('eln_g', params['eln_g'], True)
    add('eln_b', params['eln_b'], True)
    for i, lp in enumerate(params['layers']):
        p = f'l{i}_'
        add(p + 'g1', lp['g1'], True)
        add(p + 'b1', lp['b1'], True)
        for nm in ('q', 'k', 'v'):
            add_p(p + nm + 'w', lp[nm]['w'])
            add_p(p + nm + 'b', lp[nm]['b'], True)
        add_p(p + 'ew', lp['e']['w'])
        add(p + 'skw', lp['skip']['w'])
        add(p + 'skb', lp['skip']['b'], True)
        for gate in ('ga', 'gf'):
            for mm in ('m1', 'm2', 'm3'):
                add(p + gate + mm + 'w', lp[gate][mm]['w'])
                add(p + gate + mm + 'b', lp[gate][mm]['b'], True)
        add(p + 'g2', lp['g2'], True)
        add(p + 'b2', lp['b2'], True)
        add(p + 'f1w', lp['f1']['w'])
        add(p + 'f1b', lp['f1']['b'], True)
        add(p + 'f2w', lp['f2']['w'])
        add(p + 'f2b', lp['f2']['b'], True)
    for nm in ('lr1', 'lr2', 'fr1', 'fr2'):
        add(nm + 'w', params[nm]['w'])
        add(nm + 'b', params[nm]['b'], True)
    add('lr3w', params['lr3']['w'])
    add('fr3w', params['fr3']['w'])
    return out


def _mlp_gate(u, v, W, pre):
    z = jnp.concatenate([u, v, u - v], axis=-1)
    h1 = _silu(jnp.dot(z, W[pre + 'm1w'], preferred_element_type=jnp.float32)
               + W[pre + 'm1b'])
    h2 = _silu(jnp.dot(h1, W[pre + 'm2w'], preferred_element_type=jnp.float32)
               + W[pre + 'm2b'])
    a = _sig(jnp.dot(h2, W[pre + 'm3w'], preferred_element_type=jnp.float32)
             + W[pre + 'm3b'])
    return a * u + (1.0 - a) * v


def _make_body(names):
    idx = {nm: k for k, nm in enumerate(names)}

    def body(at_ref, t_ref, fr_ref, lrp_ref, *rest):
        wrefs = rest[:len(names)]
        out_lat_ref, out_fc_ref = rest[len(names):]
        W = {nm: wrefs[idx[nm]][...] for nm in names}

        f32 = jnp.float32

        tcol = at_ref[...]
        zi = jax.lax.broadcasted_iota(jnp.int32, (TN, MAXZ), 1)
        oh = (tcol - 1 == zi).astype(f32)
        h_type = jnp.dot(oh, W['emb'], preferred_element_type=f32)
        tb = jnp.broadcast_to(t_ref[...][:, None, :],
                              (G, NPER, TIME)).reshape(TN, TIME)
        h = jnp.dot(jnp.concatenate([h_type, tb], axis=-1), W['na_w'],
                    preferred_element_type=f32) + W['na_b']

        fr = fr_ref[...]
        freqs = 2.0 * jnp.pi * jax.lax.broadcasted_iota(
            jnp.int32, (1, NF), 1).astype(f32)
        angn = jnp.concatenate(
            [fr[:, d:d + 1] * freqs for d in range(3)], axis=-1)
        sn = jnp.sin(angn)
        cn = jnp.cos(angn)
        pj = jnp.concatenate([sn, cn], axis=-1).reshape(G, NPER, 60)
        qi = jnp.concatenate([cn, cn], axis=-1).reshape(G, NPER, 60)
        rj = jnp.concatenate([cn, sn], axis=-1).reshape(G, NPER, 60)
        ti = jnp.concatenate([sn, -sn], axis=-1).reshape(G, NPER, 60)
        trig = (pj[:, None, :, :] * qi[:, :, None, :]
                - rj[:, None, :, :] * ti[:, :, None, :])
        lrb = jnp.broadcast_to(lrp_ref[...][:, None, None, :],
                               (G, NPER, NPER, 6))
        ef = jnp.concatenate([trig, lrb], axis=-1)
        ef = _ln(ef, W['eln_g'], W['eln_b'])
        ef2 = ef.reshape(G * NPER * NPER, EDIM)

        drow = jax.lax.broadcasted_iota(jnp.int32, (HEADS * HID, HID), 0)
        dcol = jax.lax.broadcasted_iota(jnp.int32, (HEADS * HID, HID), 1)
        hmean = (drow // HEADS == dcol).astype(f32) / float(HEADS)

        for li in range(NL):
            p = f'l{li}_'
            h = _ln(h, W[p + 'g1'], W[p + 'b1'])
            q = jnp.dot(h, W[p + 'qw'], preferred_element_type=f32) + W[p + 'qb']
            k = jnp.dot(h, W[p + 'kw'], preferred_element_type=f32) + W[p + 'kb']
            v = jnp.dot(h, W[p + 'vw'], preferred_element_type=f32) + W[p + 'vb']
            ee = jnp.dot(ef2, W[p + 'ew'], preferred_element_type=f32)

            q3 = q.reshape(G, NPER, HEADS * HID)
            k3 = k.reshape(G, NPER, HEADS * HID)
            v3 = v.reshape(G, NPER, HEADS * HID)
            ee4 = ee.reshape(G, NPER, NPER, HEADS * HID)
            kj = k3[:, :, None, :] + ee4
            tt = q3[:, None, :, :] * kj
            a4 = tt
            width = HEADS * HID
            while width > HEADS:
                width //= 2
                a4 = a4[..., :width] + a4[..., width:]
            a4 = a4 * 0.125
            mx = jnp.max(a4, axis=1, keepdims=True)
            ex = jnp.exp(a4 - mx)
            s = jnp.sum(ex, axis=1, keepdims=True)
            al = ex / (s + 1e-16)
            al_big = al
            while al_big.shape[-1] < HEADS * HID:
                al_big = jnp.concatenate([al_big, al_big], axis=-1)
            msg = (v3[:, :, None, :] + ee4) * al_big
            out = jnp.sum(msg, axis=1)
            attn = jnp.dot(out.reshape(TN, HEADS * HID), hmean,
                           preferred_element_type=f32,
                           precision=jax.lax.Precision.HIGHEST)
            vat = attn + jnp.dot(h, W[p + 'skw'],
                                 preferred_element_type=f32) + W[p + 'skb']
            h = _mlp_gate(h, vat, W, p + 'ga')
            h = _ln(h, W[p + 'g2'], W[p + 'b2'])
            ffn = jnp.dot(_silu(jnp.dot(h, W[p + 'f1w'],
                                        preferred_element_type=f32)
                                + W[p + 'f1b']),
                          W[p + 'f2w'], preferred_element_type=f32) + W[p + 'f2b']
            h = _mlp_gate(h, ffn, W, p + 'gf')

        gf = jnp.sum(h.reshape(G, NPER, HID), axis=1) / float(NPER)
        lat = _silu(jnp.dot(gf, W['lr1w'], preferred_element_type=f32)
                    + W['lr1b'])
        lat = _silu(jnp.dot(lat, W['lr2w'], preferred_element_type=f32)
                    + W['lr2b'])
        lat = jnp.dot(lat, W['lr3w'], preferred_element_type=f32)
        fc = _silu(jnp.dot(h, W['fr1w'], preferred_element_type=f32)
                   + W['fr1b'])
        fc = _silu(jnp.dot(fc, W['fr2w'], preferred_element_type=f32)
                   + W['fr2b'])
        fc = jnp.dot(fc, W['fr3w'], preferred_element_type=f32)
        out_lat_ref[...] = lat
        out_fc_ref[...] = fc

    return body


def kernel(t, atom_types, frac_coords, lattices_rep, num_atoms, node2graph,
           lattices_mat, cemb, guide_indicator, params):
    wl = _weight_list(params)
    names = [nm for nm, _ in wl]
    warrs = [a.astype(jnp.float32) for _, a in wl]

    at2 = atom_types.reshape(N, 1)

    in_specs = [
        pl.BlockSpec((TN, 1), lambda i: (i, 0)),
        pl.BlockSpec((G, TIME), lambda i: (i, 0)),
        pl.BlockSpec((TN, 3), lambda i: (i, 0)),
        pl.BlockSpec((G, 6), lambda i: (i, 0)),
    ]
    for a in warrs:
        in_specs.append(pl.BlockSpec(a.shape, lambda i: (0, 0)))

    out_shape = (
        jax.ShapeDtypeStruct((B, 6), jnp.float32),
        jax.ShapeDtypeStruct((N, 3), jnp.float32),
    )
    out_specs = (
        pl.BlockSpec((G, 6), lambda i: (i, 0)),
        pl.BlockSpec((TN, 3), lambda i: (i, 0)),
    )

    lat, fc = pl.pallas_call(
        _make_body(names),
        grid=(NT,),
        in_specs=in_specs,
        out_specs=out_specs,
        out_shape=out_shape,
    )(at2, t, frac_coords, lattices_rep, *warrs)
    return (lat, fc)


# speedup vs baseline: 3.0113x; 1.3119x over previous
import jax
import jax.numpy as jnp
from jax.experimental import pallas as pl

B = 512
NPER = 16
N = B * NPER
HID = 64
TIME = 64
NF = 10
NL = 2
HEADS = 8
MAXZ = 100
EDIM = NF * 2 * 3 + 6

G = 16
TN = G * NPER
NT = B // G


def _sig(x):
    return 1.0 / (1.0 + jnp.exp(-x))


def _silu(x):
    return x * _sig(x)


def _ln(x, g, b):
    m = jnp.mean(x, axis=-1, keepdims=True)
    v = jnp.mean((x - m) ** 2, axis=-1, keepdims=True)
    return (x - m) / jnp.sqrt(v + 1e-5) * g + b


def _weight_list(params):
    ll = jnp.arange(HEADS * HID)
    hperm = (ll % HEADS) * HID + ll // HEADS
    out = []

    def add(name, a, as_row=False):
        if as_row:
            a = a.reshape(1, -1)
        out.append((name, a))

    def add_p(name, a, as_row=False):
        a = a[hperm] if a.ndim == 1 else a[:, hperm]
        add(name, a, as_row)

    add('emb', params['emb'])
    add('na_w', params['na']['w'])
    add('na_b', params['na']['b'], True)
    add('eln_g', params['eln_g'], True)
    add('eln_b', params['eln_b'], True)
    for i, lp in enumerate(params['layers']):
        p = f'l{i}_'
        add(p + 'g1', lp['g1'], True)
        add(p + 'b1', lp['b1'], True)
        for nm in ('q', 'k', 'v'):
            add_p(p + nm + 'w', lp[nm]['w'])
            add_p(p + nm + 'b', lp[nm]['b'], True)
        add_p(p + 'ew', lp['e']['w'])
        add(p + 'skw', lp['skip']['w'])
        add(p + 'skb', lp['skip']['b'], True)
        for gate in ('ga', 'gf'):
            for mm in ('m1', 'm2', 'm3'):
                add(p + gate + mm + 'w', lp[gate][mm]['w'])
                add(p + gate + mm + 'b', lp[gate][mm]['b'], True)
        add(p + 'g2', lp['g2'], True)
        add(p + 'b2', lp['b2'], True)
        add(p + 'f1w', lp['f1']['w'])
        add(p + 'f1b', lp['f1']['b'], True)
        add(p + 'f2w', lp['f2']['w'])
        add(p + 'f2b', lp['f2']['b'], True)
    for nm in ('lr1', 'lr2', 'fr1', 'fr2'):
        add(nm + 'w', params[nm]['w'])
        add(nm + 'b', params[nm]['b'], True)
    add('lr3w', params['lr3']['w'])
    add('fr3w', params['fr3']['w'])
    return out


def _mlp_gate(u, v, W, pre):
    z = jnp.concatenate([u, v, u - v], axis=-1)
    h1 = _silu(jnp.dot(z, W[pre + 'm1w'], preferred_element_type=jnp.float32)
               + W[pre + 'm1b'])
    h2 = _silu(jnp.dot(h1, W[pre + 'm2w'], preferred_element_type=jnp.float32)
               + W[pre + 'm2b'])
    a = _sig(jnp.dot(h2, W[pre + 'm3w'], preferred_element_type=jnp.float32)
             + W[pre + 'm3b'])
    return a * u + (1.0 - a) * v


def _make_body(names):
    idx = {nm: k for k, nm in enumerate(names)}

    def body(at_ref, t_ref, fr_ref, lrp_ref, *rest):
        wrefs = rest[:len(names)]
        out_lat_ref, out_fc_ref = rest[len(names):]
        W = {nm: wrefs[idx[nm]][...] for nm in names}

        f32 = jnp.float32

        tcol = at_ref[...]
        zi = jax.lax.broadcasted_iota(jnp.int32, (TN, MAXZ), 1)
        oh = (tcol - 1 == zi).astype(f32)
        h_type = jnp.dot(oh, W['emb'], preferred_element_type=f32)
        tb = jnp.broadcast_to(t_ref[...][:, None, :],
                              (G, NPER, TIME)).reshape(TN, TIME)
        h = jnp.dot(jnp.concatenate([h_type, tb], axis=-1), W['na_w'],
                    preferred_element_type=f32) + W['na_b']

        fr = fr_ref[...]
        freqs = 2.0 * jnp.pi * jax.lax.broadcasted_iota(
            jnp.int32, (1, NF), 1).astype(f32)
        angn = jnp.concatenate(
            [fr[:, d:d + 1] * freqs for d in range(3)], axis=-1)
        sn = jnp.sin(angn)
        cn = jnp.cos(angn)
        pj = jnp.concatenate([sn, cn], axis=-1).reshape(G, NPER, 60)
        qi = jnp.concatenate([cn, cn], axis=-1).reshape(G, NPER, 60)
        rj = jnp.concatenate([cn, sn], axis=-1).reshape(G, NPER, 60)
        ti = jnp.concatenate([sn, -sn], axis=-1).reshape(G, NPER, 60)
        trig = (pj[:, None, :, :] * qi[:, :, None, :]
                - rj[:, None, :, :] * ti[:, :, None, :])
        lrb = jnp.broadcast_to(lrp_ref[...][:, None, None, :],
                               (G, NPER, NPER, 6))
        ef = jnp.concatenate([trig, lrb], axis=-1)
        ef = _ln(ef, W['eln_g'], W['eln_b'])
        ef2 = ef.reshape(G * NPER * NPER, EDIM)

        drow = jax.lax.broadcasted_iota(jnp.int32, (HEADS * HID, HID), 0)
        dcol = jax.lax.broadcasted_iota(jnp.int32, (HEADS * HID, HID), 1)
        hmean = (drow // HEADS == dcol).astype(f32) / float(HEADS)

        for li in range(NL):
            p = f'l{li}_'
            h = _ln(h, W[p + 'g1'], W[p + 'b1'])
            q = jnp.dot(h, W[p + 'qw'], preferred_element_type=f32) + W[p + 'qb']
            k = jnp.dot(h, W[p + 'kw'], preferred_element_type=f32) + W[p + 'kb']
            v = jnp.dot(h, W[p + 'vw'], preferred_element_type=f32) + W[p + 'vb']
            ee = jnp.dot(ef2, W[p + 'ew'], preferred_element_type=f32)

            q3 = q.reshape(G, NPER, HEADS * HID)
            k3 = k.reshape(G, NPER, HEADS * HID)
            v3 = v.reshape(G, NPER, HEADS * HID)
            ee4 = ee.reshape(G, NPER, NPER, HEADS * HID)
            kj = k3[:, :, None, :] + ee4
            tt = q3[:, None, :, :] * kj
            a4 = tt
            width = HEADS * HID
            while width > HEADS:
                width //= 2
                a4 = a4[..., :width] + a4[..., width:]
            a4 = a4 * 0.125
            mx = jnp.max(a4, axis=1, keepdims=True)
            ex = jnp.exp(a4 - mx)
            s = jnp.sum(ex, axis=1, keepdims=True)
            al = ex / (s + 1e-16)
            al_big = al
            while al_big.shape[-1] < HEADS * HID:
                al_big = jnp.concatenate([al_big, al_big], axis=-1)
            msg = (v3[:, :, None, :] + ee4) * al_big
            out = jnp.sum(msg, axis=1)
            attn = jnp.dot(out.reshape(TN, HEADS * HID), hmean,
                           preferred_element_type=f32,
                           precision=jax.lax.Precision.HIGHEST)
            vat = attn + jnp.dot(h, W[p + 'skw'],
                                 preferred_element_type=f32) + W[p + 'skb']
            h = _mlp_gate(h, vat, W, p + 'ga')
            h = _ln(h, W[p + 'g2'], W[p + 'b2'])
            ffn = jnp.dot(_silu(jnp.dot(h, W[p + 'f1w'],
                                        preferred_element_type=f32)
                                + W[p + 'f1b']),
                          W[p + 'f2w'], preferred_element_type=f32) + W[p + 'f2b']
            h = _mlp_gate(h, ffn, W, p + 'gf')

        gf = jnp.sum(h.reshape(G, NPER, HID), axis=1) / float(NPER)
        lat = _silu(jnp.dot(gf, W['lr1w'], preferred_element_type=f32)
                    + W['lr1b'])
        lat = _silu(jnp.dot(lat, W['lr2w'], preferred_element_type=f32)
                    + W['lr2b'])
        lat = jnp.dot(lat, W['lr3w'], preferred_element_type=f32)
        fc = _silu(jnp.dot(h, W['fr1w'], preferred_element_type=f32)
                   + W['fr1b'])
        fc = _silu(jnp.dot(fc, W['fr2w'], preferred_element_type=f32)
                   + W['fr2b'])
        fc = jnp.dot(fc, W['fr3w'], preferred_element_type=f32)
        out_lat_ref[...] = lat
        out_fc_ref[...] = fc

    return body


def kernel(t, atom_types, frac_coords, lattices_rep, num_atoms, node2graph,
           lattices_mat, cemb, guide_indicator, params):
    wl = _weight_list(params)
    names = [nm for nm, _ in wl]
    warrs = [a.astype(jnp.float32) for _, a in wl]

    at2 = atom_types.reshape(N, 1)

    in_specs = [
        pl.BlockSpec((TN, 1), lambda i: (i, 0)),
        pl.BlockSpec((G, TIME), lambda i: (i, 0)),
        pl.BlockSpec((TN, 3), lambda i: (i, 0)),
        pl.BlockSpec((G, 6), lambda i: (i, 0)),
    ]
    for a in warrs:
        in_specs.append(pl.BlockSpec(a.shape, lambda i: (0, 0)))

    out_shape = (
        jax.ShapeDtypeStruct((B, 6), jnp.float32),
        jax.ShapeDtypeStruct((N, 3), jnp.float32),
    )
    out_specs = (
        pl.BlockSpec((G, 6), lambda i: (i, 0)),
        pl.BlockSpec((TN, 3), lambda i: (i, 0)),
    )

    lat, fc = pl.pallas_call(
        _make_body(names),
        grid=(NT,),
        in_specs=in_specs,
        out_specs=out_specs,
        out_shape=out_shape,
    )(at2, t, frac_coords, lattices_rep, *warrs)
    return (lat, fc)


# speedup vs baseline: 3.0762x; 1.0215x over previous
import jax
import jax.numpy as jnp
from jax.experimental import pallas as pl

B = 512
NPER = 16
N = B * NPER
HID = 64
TIME = 64
NF = 10
NL = 2
HEADS = 8
MAXZ = 100
EDIM = NF * 2 * 3 + 6

G = 16
TN = G * NPER
NT = B // G


def _sig(x):
    return 1.0 / (1.0 + jnp.exp(-x))


def _silu(x):
    return x * _sig(x)


def _ln(x, g, b):
    m = jnp.mean(x, axis=-1, keepdims=True)
    v = jnp.mean((x - m) ** 2, axis=-1, keepdims=True)
    return (x - m) / jnp.sqrt(v + 1e-5) * g + b


def _weight_list(params):
    ll = jnp.arange(HEADS * HID)
    hperm = (ll % HEADS) * HID + ll // HEADS
    out = []

    def add(name, a, as_row=False):
        if as_row:
            a = a.reshape(1, -1)
        out.append((name, a))

    def add_p(name, a, as_row=False):
        a = a[hperm] if a.ndim == 1 else a[:, hperm]
        add(name, a, as_row)

    add('emb', params['emb'])
    add('na_w', params['na']['w'])
    add('na_b', params['na']['b'], True)
    eln_g = params['eln_g']
    eln_b = params['eln_b']
    for i, lp in enumerate(params['layers']):
        p = f'l{i}_'
        add(p + 'g1', lp['g1'], True)
        add(p + 'b1', lp['b1'], True)
        for nm in ('q', 'k', 'v'):
            add_p(p + nm + 'w', lp[nm]['w'])
            add_p(p + nm + 'b', lp[nm]['b'], True)
        ew_aug = jnp.concatenate(
            [eln_g[:, None] * lp['e']['w'],
             (eln_b @ lp['e']['w'])[None, :]], axis=0)
        add_p(p + 'ew', ew_aug)
        add(p + 'skw', lp['skip']['w'])
        add(p + 'skb', lp['skip']['b'], True)
        for gate in ('ga', 'gf'):
            for mm in ('m1', 'm2', 'm3'):
                add(p + gate + mm + 'w', lp[gate][mm]['w'])
                add(p + gate + mm + 'b', lp[gate][mm]['b'], True)
        add(p + 'g2', lp['g2'], True)
        add(p + 'b2', lp['b2'], True)
        add(p + 'f1w', lp['f1']['w'])
        add(p + 'f1b', lp['f1']['b'], True)
        add(p + 'f2w', lp['f2']['w'])
        add(p + 'f2b', lp['f2']['b'], True)
    for nm in ('lr1', 'lr2', 'fr1', 'fr2'):
        add(nm + 'w', params[nm]['w'])
        add(nm + 'b', params[nm]['b'], True)
    add('lr3w', params['lr3']['w'])
    add('fr3w', params['fr3']['w'])
    return out


def _mlp_gate(u, v, W, pre):
    z = jnp.concatenate([u, v, u - v], axis=-1)
    h1 = _silu(jnp.dot(z, W[pre + 'm1w'], preferred_element_type=jnp.float32)
               + W[pre + 'm1b'])
    h2 = _silu(jnp.dot(h1, W[pre + 'm2w'], preferred_element_type=jnp.float32)
               + W[pre + 'm2b'])
    a = _sig(jnp.dot(h2, W[pre + 'm3w'], preferred_element_type=jnp.float32)
             + W[pre + 'm3b'])
    return a * u + (1.0 - a) * v


def _make_body(names):
    idx = {nm: k for k, nm in enumerate(names)}

    def body(at_ref, t_ref, fr_ref, lrp_ref, *rest):
        wrefs = rest[:len(names)]
        out_lat_ref, out_fc_ref = rest[len(names):]
        W = {nm: wrefs[idx[nm]][...] for nm in names}

        f32 = jnp.float32

        tcol = at_ref[...]
        zi = jax.lax.broadcasted_iota(jnp.int32, (TN, MAXZ), 1)
        oh = (tcol - 1 == zi).astype(f32)
        h_type = jnp.dot(oh, W['emb'], preferred_element_type=f32)
        tb = jnp.broadcast_to(t_ref[...][:, None, :],
                              (G, NPER, TIME)).reshape(TN, TIME)
        h = jnp.dot(jnp.concatenate([h_type, tb], axis=-1), W['na_w'],
                    preferred_element_type=f32) + W['na_b']

        fr = fr_ref[...]
        freqs = 2.0 * jnp.pi * jax.lax.broadcasted_iota(
            jnp.int32, (1, NF), 1).astype(f32)
        angn = jnp.concatenate(
            [fr[:, d:d + 1] * freqs for d in range(3)], axis=-1)
        sn = jnp.sin(angn)
        cn = jnp.cos(angn)
        pj = jnp.concatenate([sn, cn], axis=-1).reshape(G, NPER, 60)
        qi = jnp.concatenate([cn, cn], axis=-1).reshape(G, NPER, 60)
        rj = jnp.concatenate([cn, sn], axis=-1).reshape(G, NPER, 60)
        ti = jnp.concatenate([sn, -sn], axis=-1).reshape(G, NPER, 60)
        trig = (pj[:, None, :, :] * qi[:, :, None, :]
                - rj[:, None, :, :] * ti[:, :, None, :])
        lrp = lrp_ref[...]
        lrb = jnp.broadcast_to(lrp[:, None, None, :],
                               (G, NPER, NPER, 6))
        ef_raw = jnp.concatenate([trig, lrb], axis=-1)
        m = jnp.mean(ef_raw, axis=-1, keepdims=True)
        ex2 = ((30.0 + jnp.sum(lrp * lrp, axis=-1, keepdims=True))
               / float(EDIM))[:, None, None, :]
        rstd = jax.lax.rsqrt(ex2 - m * m + 1e-5)
        efn = (ef_raw - m) * rstd
        ones = jnp.ones((G, NPER, NPER, 1), f32)
        ef2 = jnp.concatenate([efn, ones], axis=-1).reshape(
            G * NPER * NPER, EDIM + 1)

        drow = jax.lax.broadcasted_iota(jnp.int32, (HEADS * HID, HID), 0)
        dcol = jax.lax.broadcasted_iota(jnp.int32, (HEADS * HID, HID), 1)
        hmean = (drow // HEADS == dcol).astype(f32) / float(HEADS)

        for li in range(NL):
            p = f'l{li}_'
            h = _ln(h, W[p + 'g1'], W[p + 'b1'])
            q = jnp.dot(h, W[p + 'qw'], preferred_element_type=f32) + W[p + 'qb']
            k = jnp.dot(h, W[p + 'kw'], preferred_element_type=f32) + W[p + 'kb']
            v = jnp.dot(h, W[p + 'vw'], preferred_element_type=f32) + W[p + 'vb']
            ee = jnp.dot(ef2, W[p + 'ew'], preferred_element_type=f32)

            q3 = q.reshape(G, NPER, HEADS * HID)
            k3 = k.reshape(G, NPER, HEADS * HID)
            v3 = v.reshape(G, NPER, HEADS * HID)
            ee4 = ee.reshape(G, NPER, NPER, HEADS * HID)
            kj = k3[:, :, None, :] + ee4
            tt = q3[:, None, :, :] * kj
            a4 = tt
            width = HEADS * HID
            while width > HEADS:
                width //= 2
                a4 = a4[..., :width] + a4[..., width:]
            a4 = a4 * 0.125
            mx = jnp.max(a4, axis=1, keepdims=True)
            ex = jnp.exp(a4 - mx)
            s = jnp.sum(ex, axis=1, keepdims=True)
            al = ex / (s + 1e-16)
            al_big = al
            while al_big.shape[-1] < HEADS * HID:
                al_big = jnp.concatenate([al_big, al_big], axis=-1)
            msg = (v3[:, :, None, :] + ee4) * al_big
            out = jnp.sum(msg, axis=1)
            attn = jnp.dot(out.reshape(TN, HEADS * HID), hmean,
                           preferred_element_type=f32,
                           precision=jax.lax.Precision.HIGHEST)
            vat = attn + jnp.dot(h, W[p + 'skw'],
                                 preferred_element_type=f32) + W[p + 'skb']
            h = _mlp_gate(h, vat, W, p + 'ga')
            h = _ln(h, W[p + 'g2'], W[p + 'b2'])
            ffn = jnp.dot(_silu(jnp.dot(h, W[p + 'f1w'],
                                        preferred_element_type=f32)
                                + W[p + 'f1b']),
                          W[p + 'f2w'], preferred_element_type=f32) + W[p + 'f2b']
            h = _mlp_gate(h, ffn, W, p + 'gf')

        gf = jnp.sum(h.reshape(G, NPER, HID), axis=1) / float(NPER)
        lat = _silu(jnp.dot(gf, W['lr1w'], preferred_element_type=f32)
                    + W['lr1b'])
        lat = _silu(jnp.dot(lat, W['lr2w'], preferred_element_type=f32)
                    + W['lr2b'])
        lat = jnp.dot(lat, W['lr3w'], preferred_element_type=f32)
        fc = _silu(jnp.dot(h, W['fr1w'], preferred_element_type=f32)
                   + W['fr1b'])
        fc = _silu(jnp.dot(fc, W['fr2w'], preferred_element_type=f32)
                   + W['fr2b'])
        fc = jnp.dot(fc, W['fr3w'], preferred_element_type=f32)
        out_lat_ref[...] = lat
        out_fc_ref[...] = fc

    return body


def kernel(t, atom_types, frac_coords, lattices_rep, num_atoms, node2graph,
           lattices_mat, cemb, guide_indicator, params):
    wl = _weight_list(params)
    names = [nm for nm, _ in wl]
    warrs = [a.astype(jnp.float32) for _, a in wl]

    at2 = atom_types.reshape(N, 1)

    in_specs = [
        pl.BlockSpec((TN, 1), lambda i: (i, 0)),
        pl.BlockSpec((G, TIME), lambda i: (i, 0)),
        pl.BlockSpec((TN, 3), lambda i: (i, 0)),
        pl.BlockSpec((G, 6), lambda i: (i, 0)),
    ]
    for a in warrs:
        in_specs.append(pl.BlockSpec(a.shape, lambda i: (0, 0)))

    out_shape = (
        jax.ShapeDtypeStruct((B, 6), jnp.float32),
        jax.ShapeDtypeStruct((N, 3), jnp.float32),
    )
    out_specs = (
        pl.BlockSpec((G, 6), lambda i: (i, 0)),
        pl.BlockSpec((TN, 3), lambda i: (i, 0)),
    )

    lat, fc = pl.pallas_call(
        _make_body(names),
        grid=(NT,),
        in_specs=in_specs,
        out_specs=out_specs,
        out_shape=out_shape,
    )(at2, t, frac_coords, lattices_rep, *warrs)
    return (lat, fc)
